# baseline (speedup 1.0000x reference)
import functools
import jax
import jax.numpy as jnp
from jax import lax
from jax.experimental import pallas as pl
from jax.experimental.pallas import tpu as pltpu
from jax.experimental.pallas import tpu_sc as plsc


def kernel(sequence, embeddings):
    batch, seq_len, feat = sequence.shape
    info = plsc.get_sparse_core_info()
    nw = info.num_cores * info.num_subcores
    rows_pw = seq_len // nw
    chunk = 128
    nchunks = rows_pw // chunk
    nc = info.num_cores

    mesh = plsc.VectorSubcoreMesh(core_axis_name="c", subcore_axis_name="s")

    @functools.partial(
        pl.kernel,
        mesh=mesh,
        out_type=jax.ShapeDtypeStruct((batch, seq_len, feat), sequence.dtype),
        scratch_types=[
            pltpu.VMEM((chunk, feat), sequence.dtype),
            pltpu.SemaphoreType.DMA,
        ],
    )
    def k(emb_hbm, out_hbm, buf, sem):
        wid = lax.axis_index("s") * nc + lax.axis_index("c")
        base = wid * rows_pw
        for c in range(nchunks):
            sl = pl.ds(base + c * chunk, chunk)
            pltpu.sync_copy(emb_hbm.at[sl, :], buf)
            cps = [
                pltpu.async_copy(buf, out_hbm.at[b, sl, :], sem)
                for b in range(batch)
            ]
            for cp in cps:
                cp.wait()

    return k(embeddings)
